# baseline (speedup 1.0000x reference)
import functools

import jax
import jax.numpy as jnp
from jax import lax
from jax.experimental import pallas as pl
from jax.experimental.pallas import tpu as pltpu
from jax.experimental.pallas import tpu_sc as plsc

_N, _E, _D, _H, _F = 2048, 32768, 1024, 512, 32
_DH = _D // 2
_DQ = _D // 4
_HALF = 1024 * _N
_PAD = 256 * _N
_TR0 = _HALF
_C1 = _HALF + _PAD
_TR1 = _C1 + _HALF
_FLATA = 2 * (_HALF + _PAD)
_BIGKEY = 1 << 30


def _k1_body(inst, Ws, bs, Wo, bo, Wm, bm, S_o, O_o, M_o):
    x = inst[...]
    S_o[...] = jax.nn.relu(jnp.dot(x, Ws[...]) + bs[...])
    O_o[...] = jax.nn.relu(jnp.dot(x, Wo[...]) + bo[...])
    M_o[...] = jnp.dot(x, Wm[...]) + bm[...]


def _k1(inst, Ws, bs, Wo, bo, Wm, bm):
    blk = 256
    grid = _N // blk
    full = lambda shape: pl.BlockSpec(shape, lambda i: (0, 0))
    return pl.pallas_call(
        _k1_body,
        grid=(grid,),
        in_specs=[
            pl.BlockSpec((blk, _D), lambda i: (i, 0)),
            full((_D, _H)), full((1, _H)),
            full((_D, _H)), full((1, _H)),
            full((_D, _DH)), full((1, _DH)),
        ],
        out_specs=[
            pl.BlockSpec((blk, _H), lambda i: (i, 0)),
            pl.BlockSpec((blk, _H), lambda i: (i, 0)),
            pl.BlockSpec((blk, _DH), lambda i: (i, 0)),
        ],
        out_shape=[
            jax.ShapeDtypeStruct((_N, _H), jnp.float32),
            jax.ShapeDtypeStruct((_N, _H), jnp.float32),
            jax.ShapeDtypeStruct((_N, _DH), jnp.float32),
        ],
    )(inst, Ws, bs, Wo, bo, Wm, bm)


def _sc_gather(S, O, src, dst, ne):
    mesh = plsc.VectorSubcoreMesh(core_axis_name="c", subcore_axis_name="s")
    nw = 32
    per_w = ne // nw
    chunk = 32

    nchunks = per_w // chunk

    @functools.partial(
        pl.kernel,
        out_type=[
            jax.ShapeDtypeStruct((ne, _H), jnp.float32),
            jax.ShapeDtypeStruct((ne, _H), jnp.float32),
        ],
        mesh=mesh,
        scratch_types=[
            pltpu.VMEM((per_w,), jnp.int32),
            pltpu.VMEM((per_w,), jnp.int32),
            pltpu.VMEM((chunk, _H), jnp.float32),
            pltpu.VMEM((chunk, _H), jnp.float32),
            pltpu.VMEM((chunk, _H), jnp.float32),
            pltpu.VMEM((chunk, _H), jnp.float32),
            pltpu.SemaphoreType.DMA, pltpu.SemaphoreType.DMA,
            pltpu.SemaphoreType.DMA, pltpu.SemaphoreType.DMA,
            pltpu.SemaphoreType.DMA, pltpu.SemaphoreType.DMA,
            pltpu.SemaphoreType.DMA, pltpu.SemaphoreType.DMA,
        ],
    )
    def k(S_h, O_h, src_h, dst_h, se_h, oe_h, sidx, didx,
          sr0, sr1, or0, or1, gs0, gs1, go0, go1, ws0, ws1, wo0, wo1):
        wid = lax.axis_index("s") * 2 + lax.axis_index("c")
        base = wid * per_w
        pltpu.sync_copy(src_h.at[pl.ds(base, per_w)], sidx)
        pltpu.sync_copy(dst_h.at[pl.ds(base, per_w)], didx)
        srow, orow = (sr0, sr1), (or0, or1)
        gs, go = (gs0, gs1), (go0, go1)
        ws, wo = (ws0, ws1), (wo0, wo1)

        def fire_g(c, b):
            cb = c * chunk
            pltpu.async_copy(S_h.at[sidx.at[pl.ds(cb, chunk)]], srow[b], gs[b])
            pltpu.async_copy(O_h.at[didx.at[pl.ds(cb, chunk)]], orow[b], go[b])

        def wait_g(c, b):
            cb = c * chunk
            pltpu.make_async_copy(S_h.at[sidx.at[pl.ds(cb, chunk)]], srow[b], gs[b]).wait()
            pltpu.make_async_copy(O_h.at[didx.at[pl.ds(cb, chunk)]], orow[b], go[b]).wait()

        def fire_w(c, b):
            cb = c * chunk
            pltpu.async_copy(srow[b], se_h.at[pl.ds(base + cb, chunk)], ws[b])
            pltpu.async_copy(orow[b], oe_h.at[pl.ds(base + cb, chunk)], wo[b])

        def wait_w(c, b):
            cb = c * chunk
            pltpu.make_async_copy(srow[b], se_h.at[pl.ds(base + cb, chunk)], ws[b]).wait()
            pltpu.make_async_copy(orow[b], oe_h.at[pl.ds(base + cb, chunk)], wo[b]).wait()

        fire_g(0, 0)

        def body(c):
            @pl.when(c >= 2)
            def _():
                wait_w(c - 1, 1)
            fire_g(c + 1, 1)
            wait_g(c, 0)
            fire_w(c, 0)
            @pl.when(c + 2 < nchunks)
            def _():
                wait_w(c, 0)
                fire_g(c + 2, 0)
            wait_g(c + 1, 1)
            fire_w(c + 1, 1)

        pl.loop(0, nchunks, step=2)(body)
        wait_w(nchunks - 2, 0)
        wait_w(nchunks - 1, 1)

    return k(S, O, src, dst)


def _k2_body(union, se, oe, surv, Wu, bu, Wg, bg, gp1_o, m_o):
    i = pl.program_id(0)
    u = jax.nn.relu(jnp.dot(union[...], Wu[...]) + bu[...])
    p = se[...] * oe[...] * u
    af = jax.nn.relu(jnp.dot(p, Wg[...]) + bg[...])
    gate = jnp.mean(af, axis=1)
    gp1_o[0, 0, :] = gate + 1.0

    @pl.when(i == 0)
    def _():
        m_o[0, 0] = 0.0

    m_o[0, 0] = jnp.maximum(m_o[0, 0], jnp.max(gate * surv[0, 0, :]))


def _k2(union, se, oe, surv, Wu, bu, Wg, bg):
    blk = 1024
    grid = union.shape[0] // blk
    full = lambda shape: pl.BlockSpec(shape, lambda i: (0, 0))
    return pl.pallas_call(
        _k2_body,
        grid=(grid,),
        in_specs=[
            pl.BlockSpec((blk, _D), lambda i: (i, 0)),
            pl.BlockSpec((blk, _H), lambda i: (i, 0)),
            pl.BlockSpec((blk, _H), lambda i: (i, 0)),
            pl.BlockSpec((1, 1, blk), lambda i: (i, 0, 0)),
            full((_D, _H)), full((1, _H)),
            full((_H, _F)), full((1, _F)),
        ],
        out_specs=[
            pl.BlockSpec((1, 1, blk), lambda i: (i, 0, 0)),
            pl.BlockSpec((1, 1), lambda i: (0, 0), memory_space=pltpu.SMEM),
        ],
        out_shape=[
            jax.ShapeDtypeStruct((grid, 1, blk), jnp.float32),
            jax.ShapeDtypeStruct((1, 1), jnp.float32),
        ],
    )(union, se, oe, surv, Wu, bu, Wg, bg)


def _sc_scatter(gp1, skey0, skey1):
    mesh = plsc.VectorSubcoreMesh(core_axis_name="c", subcore_axis_name="s")
    rows_w = 16
    zwords = 16384
    per_core = _HALF + _PAD
    zchunks = per_core // (16 * zwords)

    @functools.partial(
        pl.kernel,
        out_type=jax.ShapeDtypeStruct((_FLATA,), jnp.float32),
        mesh=mesh,
        scratch_types=[
            pltpu.VMEM((zwords,), jnp.float32),
            pltpu.VMEM((rows_w, 128), jnp.int32),
            pltpu.VMEM((rows_w, 128), jnp.float32),
            pltpu.SemaphoreType.DMA,
            pltpu.SemaphoreType.DMA,
        ],
    )
    def k(gp1_h, skey0_h, skey1_h, a_h, zbuf, keys, vals, zsem, sem):
        core = lax.axis_index("c")
        sub = lax.axis_index("s")

        @pl.when(core == 0)
        def _():
            def zvec(i):
                zbuf[pl.ds(i * 16, 16)] = jnp.zeros((16,), jnp.float32)
            pl.loop(0, zwords // 16)(zvec)

            zbase = sub * (2 * zchunks * zwords)

            def zout(j):
                pltpu.sync_copy(zbuf, a_h.at[pl.ds(zbase + j * zwords, zwords)])
            pl.loop(0, 2 * zchunks)(zout)

        plsc.subcore_barrier()

        @pl.when(core == 0)
        def _():
            rbase = sub * rows_w
            pltpu.sync_copy(gp1_h.at[pl.ds(rbase, rows_w)], vals)
            pltpu.sync_copy(skey0_h.at[pl.ds(rbase, rows_w)], keys)

            def scat(j):
                pltpu.async_copy(vals.at[j], a_h.at[keys.at[j]], sem).wait()
            pl.loop(0, rows_w)(scat)

    return k(gp1, skey0, skey1)


def _k4_body(A, Mmsg, m, out1_o, out2_o, valid_o):
    i = pl.program_id(0)
    a = A[...]
    mask = (a != 0.0).astype(jnp.float32)
    expw = jnp.exp((a - 1.0) - m[0, 0]) * mask
    rowsum = jnp.sum(expw, axis=1, keepdims=True)
    atten = expw / (rowsum + 1e-6)
    out1_o[...] = jnp.dot(atten, Mmsg[...])
    mrow = Mmsg[pl.ds(i * 256, 256), :]

    @pl.when(i == 0)
    def _():
        out2_o[...] = jnp.zeros_like(out2_o)

    out2_o[...] += lax.dot_general(atten, mrow, (((0,), (0,)), ((), ())))
    valid_o[...] = (rowsum > 0.0).astype(jnp.float32)


def _k4(A, Mmsg, m):
    blk = 256
    grid = _N // blk
    return pl.pallas_call(
        _k4_body,
        grid=(grid,),
        in_specs=[
            pl.BlockSpec((blk, _N), lambda i: (jnp.where(i >= 4, i + 1, i), 0)),
            pl.BlockSpec((_N, _DH), lambda i: (0, 0)),
            pl.BlockSpec((1, 1), lambda i: (0, 0), memory_space=pltpu.SMEM),
        ],
        out_specs=[
            pl.BlockSpec((blk, _DH), lambda i: (i, 0)),
            pl.BlockSpec((_N, _DH), lambda i: (0, 0)),
            pl.BlockSpec((blk, 1), lambda i: (i, 0)),
        ],
        out_shape=[
            jax.ShapeDtypeStruct((_N, _DH), jnp.float32),
            jax.ShapeDtypeStruct((_N, _DH), jnp.float32),
            jax.ShapeDtypeStruct((_N, 1), jnp.float32),
        ],
    )(A, Mmsg, m)


def _k5_body(o1, o2, valid, W1, b1, gm, bt, W2, b2, out_o):
    mf = jnp.concatenate([o1[...], o2[...]], axis=1)
    h = jnp.dot(mf, W1[...]) + b1[...]
    mu = jnp.mean(h, axis=1, keepdims=True)
    var = jnp.mean((h - mu) ** 2, axis=1, keepdims=True)
    h = (h - mu) / jnp.sqrt(var + 1e-5) * gm[...] + bt[...]
    h = jax.nn.relu(h)
    out = jax.nn.relu(jnp.dot(h, W2[...]) + b2[...])
    out_o[...] = out * valid[...]


def _k5(o1, o2, valid, W1, b1, gm, bt, W2, b2):
    blk = 256
    grid = _N // blk
    full = lambda shape: pl.BlockSpec(shape, lambda i: (0, 0))
    return pl.pallas_call(
        _k5_body,
        grid=(grid,),
        in_specs=[
            pl.BlockSpec((blk, _DH), lambda i: (i, 0)),
            pl.BlockSpec((blk, _DH), lambda i: (i, 0)),
            pl.BlockSpec((blk, 1), lambda i: (i, 0)),
            full((_D, _DQ)), full((1, _DQ)),
            full((1, _DQ)), full((1, _DQ)),
            full((_DQ, _H)), full((1, _H)),
        ],
        out_specs=pl.BlockSpec((blk, _H), lambda i: (i, 0)),
        out_shape=jax.ShapeDtypeStruct((_N, _H), jnp.float32),
    )(o1, o2, valid, W1, b1, gm, bt, W2, b2)


def kernel(inst_features, union_features, rel_pair_idx, Ws, bs, Wo, bo, Wu,
           bu, Wg, bg, Wmsg, bmsg, Wout1, bout1, gamma, beta, Wout2, bout2):
    src = rel_pair_idx[:, 0]
    dst = rel_pair_idx[:, 1]
    e = jnp.arange(_E, dtype=jnp.int32)
    key = src * _N + dst
    winner = jnp.zeros((_N * _N,), jnp.int32).at[key].set(e)
    surv = winner[key] == e
    trash_off = (e & 8191) * 8
    shifted = jnp.where(key >= _HALF, key + _PAD, key)
    skey0 = jnp.where(surv, shifted, _TR0 + trash_off)
    skey1 = skey0

    S, O, Mmsg2 = _k1(inst_features, Ws, bs.reshape(1, -1), Wo,
                      bo.reshape(1, -1), Wmsg, bmsg.reshape(1, -1))
    eh = _E // 2
    survf = surv.astype(jnp.float32).reshape(_E // 1024, 1, 1024)
    gp1s, ms = [], []
    for h in range(2):
        sl = slice(h * eh, (h + 1) * eh)
        Se, Oe = _sc_gather(S, O, src[sl], dst[sl], eh)
        gp1_h, m_h = _k2(union_features[sl], Se, Oe, survf[h * eh // 1024:
                         (h + 1) * eh // 1024],
                         Wu, bu.reshape(1, -1), Wg, bg.reshape(1, -1))
        gp1s.append(gp1_h.reshape(eh))
        ms.append(m_h)
    gp1 = jnp.concatenate(gp1s)
    m = jnp.maximum(ms[0], ms[1])
    A_flat = _sc_scatter(gp1.reshape(256, 128), skey0.reshape(256, 128),
                         skey1.reshape(256, 128))
    A2d = A_flat.reshape(_FLATA // _N, _N)
    out1, out2, valid = _k4(A2d, Mmsg2, m)
    return _k5(out1, out2, valid, Wout1, bout1.reshape(1, -1),
               gamma.reshape(1, -1), beta.reshape(1, -1), Wout2,
               bout2.reshape(1, -1))


# speedup vs baseline: 1.3055x; 1.3055x over previous
import functools

import jax
import jax.numpy as jnp
from jax import lax
from jax.experimental.compute_on import compute_on
from jax.experimental import pallas as pl
from jax.experimental.pallas import tpu as pltpu
from jax.experimental.pallas import tpu_sc as plsc

_N, _E, _D, _H, _F = 2048, 32768, 1024, 512, 32
_DH = _D // 2
_DQ = _D // 4
_HALF = 1024 * _N
_PAD = 256 * _N
_TR0 = _HALF
_C1 = _HALF + _PAD
_TR1 = _C1 + _HALF
_FLATA = 2 * (_HALF + _PAD)
_BIGKEY = 1 << 30


def _k1_body(inst, Ws, bs, Wo, bo, Wm, bm, S_o, O_o, M_o):
    x = inst[...]
    S_o[...] = jax.nn.relu(jnp.dot(x, Ws[...]) + bs[...])
    O_o[...] = jax.nn.relu(jnp.dot(x, Wo[...]) + bo[...])
    M_o[...] = jnp.dot(x, Wm[...]) + bm[...]


def _k1(inst, Ws, bs, Wo, bo, Wm, bm):
    blk = 256
    grid = _N // blk
    full = lambda shape: pl.BlockSpec(shape, lambda i: (0, 0))
    return pl.pallas_call(
        _k1_body,
        grid=(grid,),
        in_specs=[
            pl.BlockSpec((blk, _D), lambda i: (i, 0)),
            full((_D, _H)), full((1, _H)),
            full((_D, _H)), full((1, _H)),
            full((_D, _DH)), full((1, _DH)),
        ],
        out_specs=[
            pl.BlockSpec((blk, _H), lambda i: (i, 0)),
            pl.BlockSpec((blk, _H), lambda i: (i, 0)),
            pl.BlockSpec((blk, _DH), lambda i: (i, 0)),
        ],
        out_shape=[
            jax.ShapeDtypeStruct((_N, _H), jnp.float32),
            jax.ShapeDtypeStruct((_N, _H), jnp.float32),
            jax.ShapeDtypeStruct((_N, _DH), jnp.float32),
        ],
    )(inst, Ws, bs, Wo, bo, Wm, bm)


def _sc_gather(S, O, src, dst, ne):
    mesh = plsc.VectorSubcoreMesh(core_axis_name="c", subcore_axis_name="s")
    nw = 32
    per_w = ne // nw
    chunk = 32

    nchunks = per_w // chunk

    @functools.partial(
        pl.kernel,
        out_type=[
            jax.ShapeDtypeStruct((ne, _H), jnp.float32),
            jax.ShapeDtypeStruct((ne, _H), jnp.float32),
        ],
        mesh=mesh,
        scratch_types=[
            pltpu.VMEM((per_w,), jnp.int32),
            pltpu.VMEM((per_w,), jnp.int32),
            pltpu.VMEM((chunk, _H), jnp.float32),
            pltpu.VMEM((chunk, _H), jnp.float32),
            pltpu.VMEM((chunk, _H), jnp.float32),
            pltpu.VMEM((chunk, _H), jnp.float32),
            pltpu.SemaphoreType.DMA, pltpu.SemaphoreType.DMA,
            pltpu.SemaphoreType.DMA, pltpu.SemaphoreType.DMA,
            pltpu.SemaphoreType.DMA, pltpu.SemaphoreType.DMA,
            pltpu.SemaphoreType.DMA, pltpu.SemaphoreType.DMA,
        ],
    )
    def k(S_h, O_h, src_h, dst_h, se_h, oe_h, sidx, didx,
          sr0, sr1, or0, or1, gs0, gs1, go0, go1, ws0, ws1, wo0, wo1):
        wid = lax.axis_index("s") * 2 + lax.axis_index("c")
        base = wid * per_w
        pltpu.sync_copy(src_h.at[pl.ds(base, per_w)], sidx)
        pltpu.sync_copy(dst_h.at[pl.ds(base, per_w)], didx)
        srow, orow = (sr0, sr1), (or0, or1)
        gs, go = (gs0, gs1), (go0, go1)
        ws, wo = (ws0, ws1), (wo0, wo1)

        def fire_g(c, b):
            cb = c * chunk
            pltpu.async_copy(S_h.at[sidx.at[pl.ds(cb, chunk)]], srow[b], gs[b])
            pltpu.async_copy(O_h.at[didx.at[pl.ds(cb, chunk)]], orow[b], go[b])

        def wait_g(c, b):
            cb = c * chunk
            pltpu.make_async_copy(S_h.at[sidx.at[pl.ds(cb, chunk)]], srow[b], gs[b]).wait()
            pltpu.make_async_copy(O_h.at[didx.at[pl.ds(cb, chunk)]], orow[b], go[b]).wait()

        def fire_w(c, b):
            cb = c * chunk
            pltpu.async_copy(srow[b], se_h.at[pl.ds(base + cb, chunk)], ws[b])
            pltpu.async_copy(orow[b], oe_h.at[pl.ds(base + cb, chunk)], wo[b])

        def wait_w(c, b):
            cb = c * chunk
            pltpu.make_async_copy(srow[b], se_h.at[pl.ds(base + cb, chunk)], ws[b]).wait()
            pltpu.make_async_copy(orow[b], oe_h.at[pl.ds(base + cb, chunk)], wo[b]).wait()

        fire_g(0, 0)

        def body(c):
            @pl.when(c >= 2)
            def _():
                wait_w(c - 1, 1)
            fire_g(c + 1, 1)
            wait_g(c, 0)
            fire_w(c, 0)
            @pl.when(c + 2 < nchunks)
            def _():
                wait_w(c, 0)
                fire_g(c + 2, 0)
            wait_g(c + 1, 1)
            fire_w(c + 1, 1)

        pl.loop(0, nchunks, step=2)(body)
        wait_w(nchunks - 2, 0)
        wait_w(nchunks - 1, 1)

    return k(S, O, src, dst)


def _k2_body(union, se, oe, surv, Wu, bu, Wg, bg, gp1_o, m_o):
    i = pl.program_id(0)
    u = jax.nn.relu(jnp.dot(union[...], Wu[...]) + bu[...])
    p = se[...] * oe[...] * u
    af = jax.nn.relu(jnp.dot(p, Wg[...]) + bg[...])
    gate = jnp.mean(af, axis=1)
    gp1_o[0, 0, :] = gate + 1.0

    @pl.when(i == 0)
    def _():
        m_o[0, 0] = 0.0

    m_o[0, 0] = jnp.maximum(m_o[0, 0], jnp.max(gate * surv[0, 0, :]))


def _k2(union, se, oe, surv, Wu, bu, Wg, bg):
    blk = 1024
    grid = union.shape[0] // blk
    full = lambda shape: pl.BlockSpec(shape, lambda i: (0, 0))
    return pl.pallas_call(
        _k2_body,
        grid=(grid,),
        in_specs=[
            pl.BlockSpec((blk, _D), lambda i: (i, 0)),
            pl.BlockSpec((blk, _H), lambda i: (i, 0)),
            pl.BlockSpec((blk, _H), lambda i: (i, 0)),
            pl.BlockSpec((1, 1, blk), lambda i: (i, 0, 0)),
            full((_D, _H)), full((1, _H)),
            full((_H, _F)), full((1, _F)),
        ],
        out_specs=[
            pl.BlockSpec((1, 1, blk), lambda i: (i, 0, 0)),
            pl.BlockSpec((1, 1), lambda i: (0, 0), memory_space=pltpu.SMEM),
        ],
        out_shape=[
            jax.ShapeDtypeStruct((grid, 1, blk), jnp.float32),
            jax.ShapeDtypeStruct((1, 1), jnp.float32),
        ],
    )(union, se, oe, surv, Wu, bu, Wg, bg)


def _sc_scatter(gp1, skey0, skey1):
    mesh = plsc.VectorSubcoreMesh(core_axis_name="c", subcore_axis_name="s")
    rows_w = 16
    zwords = 16384
    per_core = _HALF + _PAD
    zchunks = per_core // (16 * zwords)

    @functools.partial(
        pl.kernel,
        out_type=jax.ShapeDtypeStruct((_FLATA,), jnp.float32),
        mesh=mesh,
        scratch_types=[
            pltpu.VMEM((zwords,), jnp.float32),
            pltpu.VMEM((rows_w, 128), jnp.int32),
            pltpu.VMEM((rows_w, 128), jnp.float32),
            pltpu.SemaphoreType.DMA,
            pltpu.SemaphoreType.DMA,
        ],
    )
    def k(gp1_h, skey0_h, skey1_h, a_h, zbuf, keys, vals, zsem, sem):
        core = lax.axis_index("c")
        sub = lax.axis_index("s")

        @pl.when(core == 0)
        def _():
            def zvec(i):
                zbuf[pl.ds(i * 16, 16)] = jnp.zeros((16,), jnp.float32)
            pl.loop(0, zwords // 16)(zvec)

            zbase = sub * (2 * zchunks * zwords)

            def zout(j):
                pltpu.sync_copy(zbuf, a_h.at[pl.ds(zbase + j * zwords, zwords)])
            pl.loop(0, 2 * zchunks)(zout)

        plsc.subcore_barrier()

        @pl.when(core == 0)
        def _():
            rbase = sub * rows_w
            pltpu.sync_copy(gp1_h.at[pl.ds(rbase, rows_w)], vals)
            pltpu.sync_copy(skey0_h.at[pl.ds(rbase, rows_w)], keys)

            def scat(j):
                pltpu.async_copy(vals.at[j], a_h.at[keys.at[j]], sem).wait()
            pl.loop(0, rows_w)(scat)

    return k(gp1, skey0, skey1)


def _k4_body(A, Mmsg, m, out1_o, out2_o, valid_o):
    i = pl.program_id(0)
    a = A[...]
    mask = (a != 0.0).astype(jnp.float32)
    expw = jnp.exp((a - 1.0) - m[0, 0]) * mask
    rowsum = jnp.sum(expw, axis=1, keepdims=True)
    atten = expw / (rowsum + 1e-6)
    out1_o[...] = jnp.dot(atten, Mmsg[...])
    mrow = Mmsg[pl.ds(i * 256, 256), :]

    @pl.when(i == 0)
    def _():
        out2_o[...] = jnp.zeros_like(out2_o)

    out2_o[...] += lax.dot_general(atten, mrow, (((0,), (0,)), ((), ())))
    valid_o[...] = (rowsum > 0.0).astype(jnp.float32)


def _k4(A, Mmsg, m):
    blk = 256
    grid = _N // blk
    return pl.pallas_call(
        _k4_body,
        grid=(grid,),
        in_specs=[
            pl.BlockSpec((blk, _N), lambda i: (jnp.where(i >= 4, i + 1, i), 0)),
            pl.BlockSpec((_N, _DH), lambda i: (0, 0)),
            pl.BlockSpec((1, 1), lambda i: (0, 0), memory_space=pltpu.SMEM),
        ],
        out_specs=[
            pl.BlockSpec((blk, _DH), lambda i: (i, 0)),
            pl.BlockSpec((_N, _DH), lambda i: (0, 0)),
            pl.BlockSpec((blk, 1), lambda i: (i, 0)),
        ],
        out_shape=[
            jax.ShapeDtypeStruct((_N, _DH), jnp.float32),
            jax.ShapeDtypeStruct((_N, _DH), jnp.float32),
            jax.ShapeDtypeStruct((_N, 1), jnp.float32),
        ],
    )(A, Mmsg, m)


def _k5_body(o1, o2, valid, W1, b1, gm, bt, W2, b2, out_o):
    mf = jnp.concatenate([o1[...], o2[...]], axis=1)
    h = jnp.dot(mf, W1[...]) + b1[...]
    mu = jnp.mean(h, axis=1, keepdims=True)
    var = jnp.mean((h - mu) ** 2, axis=1, keepdims=True)
    h = (h - mu) / jnp.sqrt(var + 1e-5) * gm[...] + bt[...]
    h = jax.nn.relu(h)
    out = jax.nn.relu(jnp.dot(h, W2[...]) + b2[...])
    out_o[...] = out * valid[...]


def _k5(o1, o2, valid, W1, b1, gm, bt, W2, b2):
    blk = 256
    grid = _N // blk
    full = lambda shape: pl.BlockSpec(shape, lambda i: (0, 0))
    return pl.pallas_call(
        _k5_body,
        grid=(grid,),
        in_specs=[
            pl.BlockSpec((blk, _DH), lambda i: (i, 0)),
            pl.BlockSpec((blk, _DH), lambda i: (i, 0)),
            pl.BlockSpec((blk, 1), lambda i: (i, 0)),
            full((_D, _DQ)), full((1, _DQ)),
            full((1, _DQ)), full((1, _DQ)),
            full((_DQ, _H)), full((1, _H)),
        ],
        out_specs=pl.BlockSpec((blk, _H), lambda i: (i, 0)),
        out_shape=jax.ShapeDtypeStruct((_N, _H), jnp.float32),
    )(o1, o2, valid, W1, b1, gm, bt, W2, b2)


def kernel(inst_features, union_features, rel_pair_idx, Ws, bs, Wo, bo, Wu,
           bu, Wg, bg, Wmsg, bmsg, Wout1, bout1, gamma, beta, Wout2, bout2):
    src = rel_pair_idx[:, 0]
    dst = rel_pair_idx[:, 1]
    e = jnp.arange(_E, dtype=jnp.int32)
    key = src * _N + dst
    @compute_on("tpu_sparsecore")
    def _winner(kk, ee):
        return jnp.zeros((_N * _N,), jnp.int32).at[kk].max(ee)

    surv = _winner(key, e)[key] == e
    trash_off = (e & 8191) * 8
    shifted = jnp.where(key >= _HALF, key + _PAD, key)
    skey0 = jnp.where(surv, shifted, _TR0 + trash_off)
    skey1 = skey0

    S, O, Mmsg2 = _k1(inst_features, Ws, bs.reshape(1, -1), Wo,
                      bo.reshape(1, -1), Wmsg, bmsg.reshape(1, -1))
    Se, Oe = _sc_gather(S, O, src, dst, _E)
    gp1, m = _k2(union_features, Se, Oe,
                 surv.astype(jnp.float32).reshape(_E // 1024, 1, 1024),
                 Wu, bu.reshape(1, -1), Wg, bg.reshape(1, -1))
    A_flat = _sc_scatter(gp1.reshape(256, 128), skey0.reshape(256, 128),
                         skey1.reshape(256, 128))
    A2d = A_flat.reshape(_FLATA // _N, _N)
    out1, out2, valid = _k4(A2d, Mmsg2, m)
    return _k5(out1, out2, valid, Wout1, bout1.reshape(1, -1),
               gamma.reshape(1, -1), beta.reshape(1, -1), Wout2,
               bout2.reshape(1, -1))


# speedup vs baseline: 1.3132x; 1.0059x over previous
import functools

import jax
import jax.numpy as jnp
from jax import lax
from jax.experimental.compute_on import compute_on
from jax.experimental import pallas as pl
from jax.experimental.pallas import tpu as pltpu
from jax.experimental.pallas import tpu_sc as plsc

_N, _E, _D, _H, _F = 2048, 32768, 1024, 512, 32
_DH = _D // 2
_DQ = _D // 4
_HALF = 1024 * _N
_PAD = 256 * _N
_TR0 = _HALF
_C1 = _HALF + _PAD
_TR1 = _C1 + _HALF
_FLATA = 2 * (_HALF + _PAD)
_BIGKEY = 1 << 30


def _k1_body(inst, Ws, bs, Wo, bo, Wm, bm, S_o, O_o, M_o):
    x = inst[...]
    S_o[...] = jax.nn.relu(jnp.dot(x, Ws[...]) + bs[...])
    O_o[...] = jax.nn.relu(jnp.dot(x, Wo[...]) + bo[...])
    M_o[...] = jnp.dot(x, Wm[...]) + bm[...]


def _k1(inst, Ws, bs, Wo, bo, Wm, bm):
    blk = 256
    grid = _N // blk
    full = lambda shape: pl.BlockSpec(shape, lambda i: (0, 0))
    return pl.pallas_call(
        _k1_body,
        grid=(grid,),
        in_specs=[
            pl.BlockSpec((blk, _D), lambda i: (i, 0)),
            full((_D, _H)), full((1, _H)),
            full((_D, _H)), full((1, _H)),
            full((_D, _DH)), full((1, _DH)),
        ],
        out_specs=[
            pl.BlockSpec((blk, _H), lambda i: (i, 0)),
            pl.BlockSpec((blk, _H), lambda i: (i, 0)),
            pl.BlockSpec((blk, _DH), lambda i: (i, 0)),
        ],
        out_shape=[
            jax.ShapeDtypeStruct((_N, _H), jnp.float32),
            jax.ShapeDtypeStruct((_N, _H), jnp.float32),
            jax.ShapeDtypeStruct((_N, _DH), jnp.float32),
        ],
    )(inst, Ws, bs, Wo, bo, Wm, bm)


def _sc_gather(S, O, src, dst, ne):
    mesh = plsc.VectorSubcoreMesh(core_axis_name="c", subcore_axis_name="s")
    nw = 32
    per_w = ne // nw
    chunk = 32

    nchunks = per_w // chunk

    @functools.partial(
        pl.kernel,
        out_type=[
            jax.ShapeDtypeStruct((ne, _H), jnp.float32),
            jax.ShapeDtypeStruct((ne, _H), jnp.float32),
        ],
        mesh=mesh,
        scratch_types=[
            pltpu.VMEM((per_w,), jnp.int32),
            pltpu.VMEM((per_w,), jnp.int32),
            pltpu.VMEM((chunk, _H), jnp.float32),
            pltpu.VMEM((chunk, _H), jnp.float32),
            pltpu.VMEM((chunk, _H), jnp.float32),
            pltpu.VMEM((chunk, _H), jnp.float32),
            pltpu.SemaphoreType.DMA, pltpu.SemaphoreType.DMA,
            pltpu.SemaphoreType.DMA, pltpu.SemaphoreType.DMA,
            pltpu.SemaphoreType.DMA, pltpu.SemaphoreType.DMA,
            pltpu.SemaphoreType.DMA, pltpu.SemaphoreType.DMA,
        ],
    )
    def k(S_h, O_h, src_h, dst_h, se_h, oe_h, sidx, didx,
          sr0, sr1, or0, or1, gs0, gs1, go0, go1, ws0, ws1, wo0, wo1):
        wid = lax.axis_index("s") * 2 + lax.axis_index("c")
        base = wid * per_w
        pltpu.sync_copy(src_h.at[pl.ds(base, per_w)], sidx)
        pltpu.sync_copy(dst_h.at[pl.ds(base, per_w)], didx)
        srow, orow = (sr0, sr1), (or0, or1)
        gs, go = (gs0, gs1), (go0, go1)
        ws, wo = (ws0, ws1), (wo0, wo1)

        def fire_g(c, b):
            cb = c * chunk
            pltpu.async_copy(S_h.at[sidx.at[pl.ds(cb, chunk)]], srow[b], gs[b])
            pltpu.async_copy(O_h.at[didx.at[pl.ds(cb, chunk)]], orow[b], go[b])

        def wait_g(c, b):
            cb = c * chunk
            pltpu.make_async_copy(S_h.at[sidx.at[pl.ds(cb, chunk)]], srow[b], gs[b]).wait()
            pltpu.make_async_copy(O_h.at[didx.at[pl.ds(cb, chunk)]], orow[b], go[b]).wait()

        def fire_w(c, b):
            cb = c * chunk
            pltpu.async_copy(srow[b], se_h.at[pl.ds(base + cb, chunk)], ws[b])
            pltpu.async_copy(orow[b], oe_h.at[pl.ds(base + cb, chunk)], wo[b])

        def wait_w(c, b):
            cb = c * chunk
            pltpu.make_async_copy(srow[b], se_h.at[pl.ds(base + cb, chunk)], ws[b]).wait()
            pltpu.make_async_copy(orow[b], oe_h.at[pl.ds(base + cb, chunk)], wo[b]).wait()

        fire_g(0, 0)

        def body(c):
            @pl.when(c >= 2)
            def _():
                wait_w(c - 1, 1)
            fire_g(c + 1, 1)
            wait_g(c, 0)
            fire_w(c, 0)
            @pl.when(c + 2 < nchunks)
            def _():
                wait_w(c, 0)
                fire_g(c + 2, 0)
            wait_g(c + 1, 1)
            fire_w(c + 1, 1)

        pl.loop(0, nchunks, step=2)(body)
        wait_w(nchunks - 2, 0)
        wait_w(nchunks - 1, 1)

    return k(S, O, src, dst)


def _k2_body(union, se, oe, surv, Wu, bu, Wg, bg, gp1_o, m_o):
    i = pl.program_id(0)
    u = jax.nn.relu(jnp.dot(union[...], Wu[...]) + bu[...])
    p = se[...] * oe[...] * u
    af = jax.nn.relu(jnp.dot(p, Wg[...]) + bg[...])
    gate = jnp.mean(af, axis=1)
    gp1_o[0, 0, :] = gate + 1.0

    @pl.when(i == 0)
    def _():
        m_o[0, 0] = 0.0

    m_o[0, 0] = jnp.maximum(m_o[0, 0], jnp.max(gate * surv[0, 0, :]))


def _k2(union, se, oe, surv, Wu, bu, Wg, bg):
    blk = 1024
    grid = union.shape[0] // blk
    full = lambda shape: pl.BlockSpec(shape, lambda i: (0, 0))
    return pl.pallas_call(
        _k2_body,
        grid=(grid,),
        in_specs=[
            pl.BlockSpec((blk, _D), lambda i: (i, 0)),
            pl.BlockSpec((blk, _H), lambda i: (i, 0)),
            pl.BlockSpec((blk, _H), lambda i: (i, 0)),
            pl.BlockSpec((1, 1, blk), lambda i: (i, 0, 0)),
            full((_D, _H)), full((1, _H)),
            full((_H, _F)), full((1, _F)),
        ],
        out_specs=[
            pl.BlockSpec((1, 1, blk), lambda i: (i, 0, 0)),
            pl.BlockSpec((1, 1), lambda i: (0, 0), memory_space=pltpu.SMEM),
        ],
        out_shape=[
            jax.ShapeDtypeStruct((grid, 1, blk), jnp.float32),
            jax.ShapeDtypeStruct((1, 1), jnp.float32),
        ],
    )(union, se, oe, surv, Wu, bu, Wg, bg)


def _sc_scatter(gp1, skey0, skey1):
    mesh = plsc.VectorSubcoreMesh(core_axis_name="c", subcore_axis_name="s")
    rows_w = 16
    zwords = 16384
    zper_w = _HALF // 8
    zchunks = zper_w // zwords

    @functools.partial(
        pl.kernel,
        out_type=jax.ShapeDtypeStruct((_FLATA,), jnp.float32),
        mesh=mesh,
        scratch_types=[
            pltpu.VMEM((zwords,), jnp.float32),
            pltpu.VMEM((rows_w, 128), jnp.int32),
            pltpu.VMEM((rows_w, 128), jnp.float32),
            pltpu.SemaphoreType.DMA,
            pltpu.SemaphoreType.DMA,
        ],
    )
    def k(gp1_h, skey0_h, skey1_h, a_h, zbuf, keys, vals, zsem, sem):
        core = lax.axis_index("c")
        sub = lax.axis_index("s")

        @pl.when(core == 0)
        def _():
            def zvec(i):
                zbuf[pl.ds(i * 16, 16)] = jnp.zeros((16,), jnp.float32)
            pl.loop(0, zwords // 16)(zvec)

            zbase = sub * zper_w + jnp.where(sub >= 8, _PAD, 0)

            def zout(j):
                pltpu.sync_copy(zbuf, a_h.at[pl.ds(zbase + j * zwords, zwords)])
            pl.loop(0, zchunks)(zout)

        plsc.subcore_barrier()

        @pl.when(core == 0)
        def _():
            rbase = sub * rows_w
            pltpu.sync_copy(gp1_h.at[pl.ds(rbase, rows_w)], vals)
            pltpu.sync_copy(skey0_h.at[pl.ds(rbase, rows_w)], keys)

            def scat(j):
                pltpu.async_copy(vals.at[j], a_h.at[keys.at[j]], sem).wait()
            pl.loop(0, rows_w)(scat)

    return k(gp1, skey0, skey1)


def _k4_body(A, Mmsg, m, out1_o, out2_o, valid_o):
    i = pl.program_id(0)
    a = A[...]
    mask = (a != 0.0).astype(jnp.float32)
    expw = jnp.exp((a - 1.0) - m[0, 0]) * mask
    rowsum = jnp.sum(expw, axis=1, keepdims=True)
    atten = expw / (rowsum + 1e-6)
    out1_o[...] = jnp.dot(atten, Mmsg[...])
    mrow = Mmsg[pl.ds(i * 256, 256), :]

    @pl.when(i == 0)
    def _():
        out2_o[...] = jnp.zeros_like(out2_o)

    out2_o[...] += lax.dot_general(atten, mrow, (((0,), (0,)), ((), ())))
    valid_o[...] = (rowsum > 0.0).astype(jnp.float32)


def _k4(A, Mmsg, m):
    blk = 256
    grid = _N // blk
    return pl.pallas_call(
        _k4_body,
        grid=(grid,),
        in_specs=[
            pl.BlockSpec((blk, _N), lambda i: (jnp.where(i >= 4, i + 1, i), 0)),
            pl.BlockSpec((_N, _DH), lambda i: (0, 0)),
            pl.BlockSpec((1, 1), lambda i: (0, 0), memory_space=pltpu.SMEM),
        ],
        out_specs=[
            pl.BlockSpec((blk, _DH), lambda i: (i, 0)),
            pl.BlockSpec((_N, _DH), lambda i: (0, 0)),
            pl.BlockSpec((blk, 1), lambda i: (i, 0)),
        ],
        out_shape=[
            jax.ShapeDtypeStruct((_N, _DH), jnp.float32),
            jax.ShapeDtypeStruct((_N, _DH), jnp.float32),
            jax.ShapeDtypeStruct((_N, 1), jnp.float32),
        ],
    )(A, Mmsg, m)


def _k5_body(o1, o2, valid, W1, b1, gm, bt, W2, b2, out_o):
    mf = jnp.concatenate([o1[...], o2[...]], axis=1)
    h = jnp.dot(mf, W1[...]) + b1[...]
    mu = jnp.mean(h, axis=1, keepdims=True)
    var = jnp.mean((h - mu) ** 2, axis=1, keepdims=True)
    h = (h - mu) / jnp.sqrt(var + 1e-5) * gm[...] + bt[...]
    h = jax.nn.relu(h)
    out = jax.nn.relu(jnp.dot(h, W2[...]) + b2[...])
    out_o[...] = out * valid[...]


def _k5(o1, o2, valid, W1, b1, gm, bt, W2, b2):
    blk = 256
    grid = _N // blk
    full = lambda shape: pl.BlockSpec(shape, lambda i: (0, 0))
    return pl.pallas_call(
        _k5_body,
        grid=(grid,),
        in_specs=[
            pl.BlockSpec((blk, _DH), lambda i: (i, 0)),
            pl.BlockSpec((blk, _DH), lambda i: (i, 0)),
            pl.BlockSpec((blk, 1), lambda i: (i, 0)),
            full((_D, _DQ)), full((1, _DQ)),
            full((1, _DQ)), full((1, _DQ)),
            full((_DQ, _H)), full((1, _H)),
        ],
        out_specs=pl.BlockSpec((blk, _H), lambda i: (i, 0)),
        out_shape=jax.ShapeDtypeStruct((_N, _H), jnp.float32),
    )(o1, o2, valid, W1, b1, gm, bt, W2, b2)


def kernel(inst_features, union_features, rel_pair_idx, Ws, bs, Wo, bo, Wu,
           bu, Wg, bg, Wmsg, bmsg, Wout1, bout1, gamma, beta, Wout2, bout2):
    src = rel_pair_idx[:, 0]
    dst = rel_pair_idx[:, 1]
    e = jnp.arange(_E, dtype=jnp.int32)
    key = src * _N + dst
    @compute_on("tpu_sparsecore")
    def _winner(kk, ee):
        return jnp.zeros((_N * _N,), jnp.int32).at[kk].max(ee)

    surv = _winner(key, e)[key] == e
    trash_off = (e & 8191) * 8
    shifted = jnp.where(key >= _HALF, key + _PAD, key)
    skey0 = jnp.where(surv, shifted, _TR0 + trash_off)
    skey1 = skey0

    S, O, Mmsg2 = _k1(inst_features, Ws, bs.reshape(1, -1), Wo,
                      bo.reshape(1, -1), Wmsg, bmsg.reshape(1, -1))
    Se, Oe = _sc_gather(S, O, src, dst, _E)
    gp1, m = _k2(union_features, Se, Oe,
                 surv.astype(jnp.float32).reshape(_E // 1024, 1, 1024),
                 Wu, bu.reshape(1, -1), Wg, bg.reshape(1, -1))
    A_flat = _sc_scatter(gp1.reshape(256, 128), skey0.reshape(256, 128),
                         skey1.reshape(256, 128))
    A2d = A_flat.reshape(_FLATA // _N, _N)
    out1, out2, valid = _k4(A2d, Mmsg2, m)
    return _k5(out1, out2, valid, Wout1, bout1.reshape(1, -1),
               gamma.reshape(1, -1), beta.reshape(1, -1), Wout2,
               bout2.reshape(1, -1))


# speedup vs baseline: 1.4740x; 1.1224x over previous
import functools

import jax
import jax.numpy as jnp
from jax import lax
from jax.experimental.compute_on import compute_on
from jax.experimental import pallas as pl
from jax.experimental.pallas import tpu as pltpu
from jax.experimental.pallas import tpu_sc as plsc

_N, _E, _D, _H, _F = 2048, 32768, 1024, 512, 32
_DH = _D // 2
_DQ = _D // 4
_HALF = 1024 * _N
_PAD = 256 * _N
_TR0 = _HALF
_C1 = _HALF + _PAD
_TR1 = _C1 + _HALF
_FLATA = 2 * (_HALF + _PAD)
_BIGKEY = 1 << 30


def _pack_bf16_pair(v):
    hi = lax.bitcast_convert_type(v[:, _DH // 2:].astype(jnp.bfloat16),
                                  jnp.uint16).astype(jnp.uint32)
    lo = lax.bitcast_convert_type(v[:, : _DH // 2].astype(jnp.bfloat16),
                                  jnp.uint16).astype(jnp.uint32)
    return lax.bitcast_convert_type((hi << 16) | lo, jnp.float32)


def _unpack_bf16_pair(w):
    u = lax.bitcast_convert_type(w, jnp.uint32)
    lo = lax.bitcast_convert_type((u & 0xFFFF).astype(jnp.uint16),
                                  jnp.bfloat16).astype(jnp.float32)
    hi = lax.bitcast_convert_type((u >> 16).astype(jnp.uint16),
                                  jnp.bfloat16).astype(jnp.float32)
    return lo, hi


def _k1_body(inst, Ws, bs, Wo, bo, Wm, bm, S_o, O_o, M_o):
    x = inst[...]
    S_o[...] = _pack_bf16_pair(jax.nn.relu(jnp.dot(x, Ws[...]) + bs[...]))
    O_o[...] = _pack_bf16_pair(jax.nn.relu(jnp.dot(x, Wo[...]) + bo[...]))
    M_o[...] = jnp.dot(x, Wm[...]) + bm[...]


def _k1(inst, Ws, bs, Wo, bo, Wm, bm):
    blk = 256
    grid = _N // blk
    full = lambda shape: pl.BlockSpec(shape, lambda i: (0, 0))
    return pl.pallas_call(
        _k1_body,
        grid=(grid,),
        in_specs=[
            pl.BlockSpec((blk, _D), lambda i: (i, 0)),
            full((_D, _H)), full((1, _H)),
            full((_D, _H)), full((1, _H)),
            full((_D, _DH)), full((1, _DH)),
        ],
        out_specs=[
            pl.BlockSpec((blk, _H // 2), lambda i: (i, 0)),
            pl.BlockSpec((blk, _H // 2), lambda i: (i, 0)),
            pl.BlockSpec((blk, _DH), lambda i: (i, 0)),
        ],
        out_shape=[
            jax.ShapeDtypeStruct((_N, _H // 2), jnp.float32),
            jax.ShapeDtypeStruct((_N, _H // 2), jnp.float32),
            jax.ShapeDtypeStruct((_N, _DH), jnp.float32),
        ],
    )(inst, Ws, bs, Wo, bo, Wm, bm)


def _sc_gather(S, O, src, dst, ne):
    mesh = plsc.VectorSubcoreMesh(core_axis_name="c", subcore_axis_name="s")
    nw = 32
    per_w = ne // nw
    chunk = 32

    nchunks = per_w // chunk

    @functools.partial(
        pl.kernel,
        out_type=[
            jax.ShapeDtypeStruct((ne, _H // 2), jnp.float32),
            jax.ShapeDtypeStruct((ne, _H // 2), jnp.float32),
        ],
        mesh=mesh,
        scratch_types=[
            pltpu.VMEM((per_w,), jnp.int32),
            pltpu.VMEM((per_w,), jnp.int32),
            pltpu.VMEM((chunk, _H // 2), jnp.float32),
            pltpu.VMEM((chunk, _H // 2), jnp.float32),
            pltpu.VMEM((chunk, _H // 2), jnp.float32),
            pltpu.VMEM((chunk, _H // 2), jnp.float32),
            pltpu.SemaphoreType.DMA, pltpu.SemaphoreType.DMA,
            pltpu.SemaphoreType.DMA, pltpu.SemaphoreType.DMA,
            pltpu.SemaphoreType.DMA, pltpu.SemaphoreType.DMA,
            pltpu.SemaphoreType.DMA, pltpu.SemaphoreType.DMA,
        ],
    )
    def k(S_h, O_h, src_h, dst_h, se_h, oe_h, sidx, didx,
          sr0, sr1, or0, or1, gs0, gs1, go0, go1, ws0, ws1, wo0, wo1):
        wid = lax.axis_index("s") * 2 + lax.axis_index("c")
        base = wid * per_w
        pltpu.sync_copy(src_h.at[pl.ds(base, per_w)], sidx)
        pltpu.sync_copy(dst_h.at[pl.ds(base, per_w)], didx)
        srow, orow = (sr0, sr1), (or0, or1)
        gs, go = (gs0, gs1), (go0, go1)
        ws, wo = (ws0, ws1), (wo0, wo1)

        def fire_g(c, b):
            cb = c * chunk
            pltpu.async_copy(S_h.at[sidx.at[pl.ds(cb, chunk)]], srow[b], gs[b])
            pltpu.async_copy(O_h.at[didx.at[pl.ds(cb, chunk)]], orow[b], go[b])

        def wait_g(c, b):
            cb = c * chunk
            pltpu.make_async_copy(S_h.at[sidx.at[pl.ds(cb, chunk)]], srow[b], gs[b]).wait()
            pltpu.make_async_copy(O_h.at[didx.at[pl.ds(cb, chunk)]], orow[b], go[b]).wait()

        def fire_w(c, b):
            cb = c * chunk
            pltpu.async_copy(srow[b], se_h.at[pl.ds(base + cb, chunk)], ws[b])
            pltpu.async_copy(orow[b], oe_h.at[pl.ds(base + cb, chunk)], wo[b])

        def wait_w(c, b):
            cb = c * chunk
            pltpu.make_async_copy(srow[b], se_h.at[pl.ds(base + cb, chunk)], ws[b]).wait()
            pltpu.make_async_copy(orow[b], oe_h.at[pl.ds(base + cb, chunk)], wo[b]).wait()

        fire_g(0, 0)

        def body(c):
            @pl.when(c >= 2)
            def _():
                wait_w(c - 1, 1)
            fire_g(c + 1, 1)
            wait_g(c, 0)
            fire_w(c, 0)
            @pl.when(c + 2 < nchunks)
            def _():
                wait_w(c, 0)
                fire_g(c + 2, 0)
            wait_g(c + 1, 1)
            fire_w(c + 1, 1)

        pl.loop(0, nchunks, step=2)(body)
        wait_w(nchunks - 2, 0)
        wait_w(nchunks - 1, 1)

    return k(S, O, src, dst)


def _k2_body(union, se, oe, surv, Wu, bu, Wg, bg, gp1_o, m_o):
    i = pl.program_id(0)
    u = jax.nn.relu(jnp.dot(union[...], Wu[...]) + bu[...])
    se_lo, se_hi = _unpack_bf16_pair(se[...])
    oe_lo, oe_hi = _unpack_bf16_pair(oe[...])
    hh = _H // 2
    p_lo = se_lo * oe_lo * u[:, :hh]
    p_hi = se_hi * oe_hi * u[:, hh:]
    af = jax.nn.relu(jnp.dot(p_lo, Wg[:hh, :]) + jnp.dot(p_hi, Wg[hh:, :])
                     + bg[...])
    gate = jnp.mean(af, axis=1)
    gp1_o[0, 0, :] = gate + 1.0

    @pl.when(i == 0)
    def _():
        m_o[0, 0] = 0.0

    m_o[0, 0] = jnp.maximum(m_o[0, 0], jnp.max(gate * surv[0, 0, :]))


def _k2(union, se, oe, surv, Wu, bu, Wg, bg):
    blk = 1024
    grid = union.shape[0] // blk
    full = lambda shape: pl.BlockSpec(shape, lambda i: (0, 0))
    return pl.pallas_call(
        _k2_body,
        grid=(grid,),
        in_specs=[
            pl.BlockSpec((blk, _D), lambda i: (i, 0)),
            pl.BlockSpec((blk, _H // 2), lambda i: (i, 0)),
            pl.BlockSpec((blk, _H // 2), lambda i: (i, 0)),
            pl.BlockSpec((1, 1, blk), lambda i: (i, 0, 0)),
            full((_D, _H)), full((1, _H)),
            full((_H, _F)), full((1, _F)),
        ],
        out_specs=[
            pl.BlockSpec((1, 1, blk), lambda i: (i, 0, 0)),
            pl.BlockSpec((1, 1), lambda i: (0, 0), memory_space=pltpu.SMEM),
        ],
        out_shape=[
            jax.ShapeDtypeStruct((grid, 1, blk), jnp.float32),
            jax.ShapeDtypeStruct((1, 1), jnp.float32),
        ],
    )(union, se, oe, surv, Wu, bu, Wg, bg)


def _sc_scatter(gp1, skey0, skey1):
    mesh = plsc.VectorSubcoreMesh(core_axis_name="c", subcore_axis_name="s")
    rows_w = 16
    zwords = 16384
    zper_w = _HALF // 8
    zchunks = zper_w // zwords

    @functools.partial(
        pl.kernel,
        out_type=jax.ShapeDtypeStruct((_FLATA,), jnp.float32),
        mesh=mesh,
        scratch_types=[
            pltpu.VMEM((zwords,), jnp.float32),
            pltpu.VMEM((rows_w, 128), jnp.int32),
            pltpu.VMEM((rows_w, 128), jnp.float32),
            pltpu.SemaphoreType.DMA,
            pltpu.SemaphoreType.DMA,
        ],
    )
    def k(gp1_h, skey0_h, skey1_h, a_h, zbuf, keys, vals, zsem, sem):
        core = lax.axis_index("c")
        sub = lax.axis_index("s")

        @pl.when(core == 0)
        def _():
            def zvec(i):
                zbuf[pl.ds(i * 16, 16)] = jnp.zeros((16,), jnp.float32)
            pl.loop(0, zwords // 16)(zvec)

            zbase = sub * zper_w + jnp.where(sub >= 8, _PAD, 0)

            def zout(j):
                pltpu.sync_copy(zbuf, a_h.at[pl.ds(zbase + j * zwords, zwords)])
            pl.loop(0, zchunks)(zout)

        plsc.subcore_barrier()

        @pl.when(core == 0)
        def _():
            rbase = sub * rows_w
            pltpu.sync_copy(gp1_h.at[pl.ds(rbase, rows_w)], vals)
            pltpu.sync_copy(skey0_h.at[pl.ds(rbase, rows_w)], keys)

            def scat(j):
                pltpu.async_copy(vals.at[j], a_h.at[keys.at[j]], sem).wait()
            pl.loop(0, rows_w)(scat)

    return k(gp1, skey0, skey1)


def _k4_body(A, Mmsg, m, out1_o, out2_o, valid_o):
    i = pl.program_id(0)
    a = A[...]
    mask = (a != 0.0).astype(jnp.float32)
    expw = jnp.exp((a - 1.0) - m[0, 0]) * mask
    rowsum = jnp.sum(expw, axis=1, keepdims=True)
    atten = expw / (rowsum + 1e-6)
    out1_o[...] = jnp.dot(atten, Mmsg[...])
    mrow = Mmsg[pl.ds(i * 256, 256), :]

    @pl.when(i == 0)
    def _():
        out2_o[...] = jnp.zeros_like(out2_o)

    out2_o[...] += lax.dot_general(atten, mrow, (((0,), (0,)), ((), ())))
    valid_o[...] = (rowsum > 0.0).astype(jnp.float32)


def _k4(A, Mmsg, m):
    blk = 256
    grid = _N // blk
    return pl.pallas_call(
        _k4_body,
        grid=(grid,),
        in_specs=[
            pl.BlockSpec((blk, _N), lambda i: (jnp.where(i >= 4, i + 1, i), 0)),
            pl.BlockSpec((_N, _DH), lambda i: (0, 0)),
            pl.BlockSpec((1, 1), lambda i: (0, 0), memory_space=pltpu.SMEM),
        ],
        out_specs=[
            pl.BlockSpec((blk, _DH), lambda i: (i, 0)),
            pl.BlockSpec((_N, _DH), lambda i: (0, 0)),
            pl.BlockSpec((blk, 1), lambda i: (i, 0)),
        ],
        out_shape=[
            jax.ShapeDtypeStruct((_N, _DH), jnp.float32),
            jax.ShapeDtypeStruct((_N, _DH), jnp.float32),
            jax.ShapeDtypeStruct((_N, 1), jnp.float32),
        ],
    )(A, Mmsg, m)


def _k5_body(o1, o2, valid, W1, b1, gm, bt, W2, b2, out_o):
    mf = jnp.concatenate([o1[...], o2[...]], axis=1)
    h = jnp.dot(mf, W1[...]) + b1[...]
    mu = jnp.mean(h, axis=1, keepdims=True)
    var = jnp.mean((h - mu) ** 2, axis=1, keepdims=True)
    h = (h - mu) / jnp.sqrt(var + 1e-5) * gm[...] + bt[...]
    h = jax.nn.relu(h)
    out = jax.nn.relu(jnp.dot(h, W2[...]) + b2[...])
    out_o[...] = out * valid[...]


def _k5(o1, o2, valid, W1, b1, gm, bt, W2, b2):
    blk = 256
    grid = _N // blk
    full = lambda shape: pl.BlockSpec(shape, lambda i: (0, 0))
    return pl.pallas_call(
        _k5_body,
        grid=(grid,),
        in_specs=[
            pl.BlockSpec((blk, _DH), lambda i: (i, 0)),
            pl.BlockSpec((blk, _DH), lambda i: (i, 0)),
            pl.BlockSpec((blk, 1), lambda i: (i, 0)),
            full((_D, _DQ)), full((1, _DQ)),
            full((1, _DQ)), full((1, _DQ)),
            full((_DQ, _H)), full((1, _H)),
        ],
        out_specs=pl.BlockSpec((blk, _H), lambda i: (i, 0)),
        out_shape=jax.ShapeDtypeStruct((_N, _H), jnp.float32),
    )(o1, o2, valid, W1, b1, gm, bt, W2, b2)


def kernel(inst_features, union_features, rel_pair_idx, Ws, bs, Wo, bo, Wu,
           bu, Wg, bg, Wmsg, bmsg, Wout1, bout1, gamma, beta, Wout2, bout2):
    src = rel_pair_idx[:, 0]
    dst = rel_pair_idx[:, 1]
    e = jnp.arange(_E, dtype=jnp.int32)
    key = src * _N + dst
    @compute_on("tpu_sparsecore")
    def _winner(kk, ee):
        return jnp.zeros((_N * _N,), jnp.int32).at[kk].max(ee)

    surv = _winner(key, e)[key] == e
    trash_off = (e & 8191) * 8
    shifted = jnp.where(key >= _HALF, key + _PAD, key)
    skey0 = jnp.where(surv, shifted, _TR0 + trash_off)
    skey1 = skey0

    S, O, Mmsg2 = _k1(inst_features, Ws, bs.reshape(1, -1), Wo,
                      bo.reshape(1, -1), Wmsg, bmsg.reshape(1, -1))
    Se, Oe = _sc_gather(S, O, src, dst, _E)
    gp1, m = _k2(union_features, Se, Oe,
                 surv.astype(jnp.float32).reshape(_E // 1024, 1, 1024),
                 Wu, bu.reshape(1, -1), Wg, bg.reshape(1, -1))
    A_flat = _sc_scatter(gp1.reshape(256, 128), skey0.reshape(256, 128),
                         skey1.reshape(256, 128))
    A2d = A_flat.reshape(_FLATA // _N, _N)
    out1, out2, valid = _k4(A2d, Mmsg2, m)
    return _k5(out1, out2, valid, Wout1, bout1.reshape(1, -1),
               gamma.reshape(1, -1), beta.reshape(1, -1), Wout2,
               bout2.reshape(1, -1))


# speedup vs baseline: 1.4754x; 1.0010x over previous
import functools

import jax
import jax.numpy as jnp
from jax import lax
from jax.experimental.compute_on import compute_on
from jax.experimental import pallas as pl
from jax.experimental.pallas import tpu as pltpu
from jax.experimental.pallas import tpu_sc as plsc

_N, _E, _D, _H, _F = 2048, 32768, 1024, 512, 32
_DH = _D // 2
_DQ = _D // 4
_HALF = 1024 * _N
_PAD = 256 * _N
_TR0 = _HALF
_C1 = _HALF + _PAD
_TR1 = _C1 + _HALF
_FLATA = 2 * (_HALF + _PAD)


def _pack_bf16_pair(v):
    hi = lax.bitcast_convert_type(v[:, _DH // 2:].astype(jnp.bfloat16),
                                  jnp.uint16).astype(jnp.uint32)
    lo = lax.bitcast_convert_type(v[:, : _DH // 2].astype(jnp.bfloat16),
                                  jnp.uint16).astype(jnp.uint32)
    return lax.bitcast_convert_type((hi << 16) | lo, jnp.float32)


def _unpack_bf16_pair(w):
    u = lax.bitcast_convert_type(w, jnp.uint32)
    lo = lax.bitcast_convert_type((u & 0xFFFF).astype(jnp.uint16),
                                  jnp.bfloat16).astype(jnp.float32)
    hi = lax.bitcast_convert_type((u >> 16).astype(jnp.uint16),
                                  jnp.bfloat16).astype(jnp.float32)
    return lo, hi


def _k1_body(inst, Ws, bs, Wo, bo, Wm, bm, S_o, O_o, M_o):
    x = inst[...]
    S_o[...] = _pack_bf16_pair(jax.nn.relu(jnp.dot(x, Ws[...]) + bs[...]))
    O_o[...] = _pack_bf16_pair(jax.nn.relu(jnp.dot(x, Wo[...]) + bo[...]))
    M_o[...] = jnp.dot(x, Wm[...]) + bm[...]


def _k1(inst, Ws, bs, Wo, bo, Wm, bm):
    blk = 256
    grid = _N // blk
    full = lambda shape: pl.BlockSpec(shape, lambda i: (0, 0))
    return pl.pallas_call(
        _k1_body,
        grid=(grid,),
        in_specs=[
            pl.BlockSpec((blk, _D), lambda i: (i, 0)),
            full((_D, _H)), full((1, _H)),
            full((_D, _H)), full((1, _H)),
            full((_D, _DH)), full((1, _DH)),
        ],
        out_specs=[
            pl.BlockSpec((blk, _H // 2), lambda i: (i, 0)),
            pl.BlockSpec((blk, _H // 2), lambda i: (i, 0)),
            pl.BlockSpec((blk, _DH), lambda i: (i, 0)),
        ],
        out_shape=[
            jax.ShapeDtypeStruct((_N, _H // 2), jnp.float32),
            jax.ShapeDtypeStruct((_N, _H // 2), jnp.float32),
            jax.ShapeDtypeStruct((_N, _DH), jnp.float32),
        ],
    )(inst, Ws, bs, Wo, bo, Wm, bm)


def _sc_gather(S, O, src, dst, ne):
    mesh = plsc.VectorSubcoreMesh(core_axis_name="c", subcore_axis_name="s")
    nw = 32
    per_w = ne // nw
    chunk = 32

    nchunks = per_w // chunk

    @functools.partial(
        pl.kernel,
        out_type=[
            jax.ShapeDtypeStruct((ne, _H // 2), jnp.float32),
            jax.ShapeDtypeStruct((ne, _H // 2), jnp.float32),
        ],
        mesh=mesh,
        scratch_types=[
            pltpu.VMEM((per_w,), jnp.int32),
            pltpu.VMEM((per_w,), jnp.int32),
            pltpu.VMEM((chunk, _H // 2), jnp.float32),
            pltpu.VMEM((chunk, _H // 2), jnp.float32),
            pltpu.VMEM((chunk, _H // 2), jnp.float32),
            pltpu.VMEM((chunk, _H // 2), jnp.float32),
            pltpu.SemaphoreType.DMA, pltpu.SemaphoreType.DMA,
            pltpu.SemaphoreType.DMA, pltpu.SemaphoreType.DMA,
            pltpu.SemaphoreType.DMA, pltpu.SemaphoreType.DMA,
            pltpu.SemaphoreType.DMA, pltpu.SemaphoreType.DMA,
        ],
    )
    def k(S_h, O_h, src_h, dst_h, se_h, oe_h, sidx, didx,
          sr0, sr1, or0, or1, gs0, gs1, go0, go1, ws0, ws1, wo0, wo1):
        wid = lax.axis_index("s") * 2 + lax.axis_index("c")
        base = wid * per_w
        pltpu.sync_copy(src_h.at[pl.ds(base, per_w)], sidx)
        pltpu.sync_copy(dst_h.at[pl.ds(base, per_w)], didx)
        srow, orow = (sr0, sr1), (or0, or1)
        gs, go = (gs0, gs1), (go0, go1)
        ws, wo = (ws0, ws1), (wo0, wo1)

        def fire_g(c, b):
            cb = c * chunk
            pltpu.async_copy(S_h.at[sidx.at[pl.ds(cb, chunk)]], srow[b], gs[b])
            pltpu.async_copy(O_h.at[didx.at[pl.ds(cb, chunk)]], orow[b], go[b])

        def wait_g(c, b):
            cb = c * chunk
            pltpu.make_async_copy(S_h.at[sidx.at[pl.ds(cb, chunk)]], srow[b], gs[b]).wait()
            pltpu.make_async_copy(O_h.at[didx.at[pl.ds(cb, chunk)]], orow[b], go[b]).wait()

        def fire_w(c, b):
            cb = c * chunk
            pltpu.async_copy(srow[b], se_h.at[pl.ds(base + cb, chunk)], ws[b])
            pltpu.async_copy(orow[b], oe_h.at[pl.ds(base + cb, chunk)], wo[b])

        def wait_w(c, b):
            cb = c * chunk
            pltpu.make_async_copy(srow[b], se_h.at[pl.ds(base + cb, chunk)], ws[b]).wait()
            pltpu.make_async_copy(orow[b], oe_h.at[pl.ds(base + cb, chunk)], wo[b]).wait()

        fire_g(0, 0)

        def body(c):
            @pl.when(c >= 2)
            def _():
                wait_w(c - 1, 1)
            fire_g(c + 1, 1)
            wait_g(c, 0)
            fire_w(c, 0)
            @pl.when(c + 2 < nchunks)
            def _():
                wait_w(c, 0)
                fire_g(c + 2, 0)
            wait_g(c + 1, 1)
            fire_w(c + 1, 1)

        pl.loop(0, nchunks, step=2)(body)
        wait_w(nchunks - 2, 0)
        wait_w(nchunks - 1, 1)

    return k(S, O, src, dst)


def _k2_body(union, se, oe, surv, Wu, bu, Wg, bg, gp1_o, m_o):
    i = pl.program_id(0)
    u = jax.nn.relu(jnp.dot(union[...], Wu[...]) + bu[...])
    se_lo, se_hi = _unpack_bf16_pair(se[...])
    oe_lo, oe_hi = _unpack_bf16_pair(oe[...])
    hh = _H // 2
    p_lo = se_lo * oe_lo * u[:, :hh]
    p_hi = se_hi * oe_hi * u[:, hh:]
    af = jax.nn.relu(jnp.dot(p_lo, Wg[:hh, :]) + jnp.dot(p_hi, Wg[hh:, :])
                     + bg[...])
    gate = jnp.mean(af, axis=1)
    gp1_o[0, 0, :] = gate + 1.0

    @pl.when(i == 0)
    def _():
        m_o[0, 0] = 0.0

    m_o[0, 0] = jnp.maximum(m_o[0, 0], jnp.max(gate * surv[0, 0, :]))


def _k2(union, se, oe, surv, Wu, bu, Wg, bg):
    blk = 1024
    grid = union.shape[0] // blk
    full = lambda shape: pl.BlockSpec(shape, lambda i: (0, 0))
    return pl.pallas_call(
        _k2_body,
        grid=(grid,),
        in_specs=[
            pl.BlockSpec((blk, _D), lambda i: (i, 0)),
            pl.BlockSpec((blk, _H // 2), lambda i: (i, 0)),
            pl.BlockSpec((blk, _H // 2), lambda i: (i, 0)),
            pl.BlockSpec((1, 1, blk), lambda i: (i, 0, 0)),
            full((_D, _H)), full((1, _H)),
            full((_H, _F)), full((1, _F)),
        ],
        out_specs=[
            pl.BlockSpec((1, 1, blk), lambda i: (i, 0, 0)),
            pl.BlockSpec((1, 1), lambda i: (0, 0), memory_space=pltpu.SMEM),
        ],
        out_shape=[
            jax.ShapeDtypeStruct((grid, 1, blk), jnp.float32),
            jax.ShapeDtypeStruct((1, 1), jnp.float32),
        ],
    )(union, se, oe, surv, Wu, bu, Wg, bg)


def _sc_scatter(gp1, skey):
    mesh = plsc.VectorSubcoreMesh(core_axis_name="c", subcore_axis_name="s")
    rows_w = 16
    zwords = 16384
    zper_w = _HALF // 8
    zchunks = zper_w // zwords

    @functools.partial(
        pl.kernel,
        out_type=jax.ShapeDtypeStruct((_FLATA,), jnp.float32),
        mesh=mesh,
        scratch_types=[
            pltpu.VMEM((zwords,), jnp.float32),
            pltpu.VMEM((rows_w, 128), jnp.int32),
            pltpu.VMEM((rows_w, 128), jnp.float32),
            pltpu.SemaphoreType.DMA,
        ],
    )
    def k(gp1_h, skey_h, a_h, zbuf, keys, vals, sem):
        core = lax.axis_index("c")
        sub = lax.axis_index("s")

        @pl.when(core == 0)
        def _():
            def zvec(i):
                zbuf[pl.ds(i * 16, 16)] = jnp.zeros((16,), jnp.float32)
            pl.loop(0, zwords // 16)(zvec)

            zbase = sub * zper_w + jnp.where(sub >= 8, _PAD, 0)

            def zout(j):
                pltpu.sync_copy(zbuf, a_h.at[pl.ds(zbase + j * zwords, zwords)])
            pl.loop(0, zchunks)(zout)

        plsc.subcore_barrier()

        @pl.when(core == 0)
        def _():
            rbase = sub * rows_w
            pltpu.sync_copy(gp1_h.at[pl.ds(rbase, rows_w)], vals)
            pltpu.sync_copy(skey_h.at[pl.ds(rbase, rows_w)], keys)

            def scat(j):
                pltpu.async_copy(vals.at[j], a_h.at[keys.at[j]], sem).wait()
            pl.loop(0, rows_w)(scat)

    return k(gp1, skey)


def _k4_body(A, Mmsg, m, out1_o, out2_o, valid_o):
    i = pl.program_id(0)
    a = A[...]
    mask = (a != 0.0).astype(jnp.float32)
    expw = jnp.exp((a - 1.0) - m[0, 0]) * mask
    rowsum = jnp.sum(expw, axis=1, keepdims=True)
    atten = expw / (rowsum + 1e-6)
    out1_o[...] = jnp.dot(atten, Mmsg[...])
    mrow = Mmsg[pl.ds(i * 256, 256), :]

    @pl.when(i == 0)
    def _():
        out2_o[...] = jnp.zeros_like(out2_o)

    out2_o[...] += lax.dot_general(atten, mrow, (((0,), (0,)), ((), ())))
    valid_o[...] = (rowsum > 0.0).astype(jnp.float32)


def _k4(A, Mmsg, m):
    blk = 256
    grid = _N // blk
    return pl.pallas_call(
        _k4_body,
        grid=(grid,),
        in_specs=[
            pl.BlockSpec((blk, _N), lambda i: (jnp.where(i >= 4, i + 1, i), 0)),
            pl.BlockSpec((_N, _DH), lambda i: (0, 0)),
            pl.BlockSpec((1, 1), lambda i: (0, 0), memory_space=pltpu.SMEM),
        ],
        out_specs=[
            pl.BlockSpec((blk, _DH), lambda i: (i, 0)),
            pl.BlockSpec((_N, _DH), lambda i: (0, 0)),
            pl.BlockSpec((blk, 1), lambda i: (i, 0)),
        ],
        out_shape=[
            jax.ShapeDtypeStruct((_N, _DH), jnp.float32),
            jax.ShapeDtypeStruct((_N, _DH), jnp.float32),
            jax.ShapeDtypeStruct((_N, 1), jnp.float32),
        ],
    )(A, Mmsg, m)


def _k5_body(o1, o2, valid, W1, b1, gm, bt, W2, b2, out_o):
    mf = jnp.concatenate([o1[...], o2[...]], axis=1)
    h = jnp.dot(mf, W1[...]) + b1[...]
    mu = jnp.mean(h, axis=1, keepdims=True)
    var = jnp.mean((h - mu) ** 2, axis=1, keepdims=True)
    h = (h - mu) / jnp.sqrt(var + 1e-5) * gm[...] + bt[...]
    h = jax.nn.relu(h)
    out = jax.nn.relu(jnp.dot(h, W2[...]) + b2[...])
    out_o[...] = out * valid[...]


def _k5(o1, o2, valid, W1, b1, gm, bt, W2, b2):
    blk = 256
    grid = _N // blk
    full = lambda shape: pl.BlockSpec(shape, lambda i: (0, 0))
    return pl.pallas_call(
        _k5_body,
        grid=(grid,),
        in_specs=[
            pl.BlockSpec((blk, _DH), lambda i: (i, 0)),
            pl.BlockSpec((blk, _DH), lambda i: (i, 0)),
            pl.BlockSpec((blk, 1), lambda i: (i, 0)),
            full((_D, _DQ)), full((1, _DQ)),
            full((1, _DQ)), full((1, _DQ)),
            full((_DQ, _H)), full((1, _H)),
        ],
        out_specs=pl.BlockSpec((blk, _H), lambda i: (i, 0)),
        out_shape=jax.ShapeDtypeStruct((_N, _H), jnp.float32),
    )(o1, o2, valid, W1, b1, gm, bt, W2, b2)


def kernel(inst_features, union_features, rel_pair_idx, Ws, bs, Wo, bo, Wu,
           bu, Wg, bg, Wmsg, bmsg, Wout1, bout1, gamma, beta, Wout2, bout2):
    src = rel_pair_idx[:, 0]
    dst = rel_pair_idx[:, 1]
    e = jnp.arange(_E, dtype=jnp.int32)
    key = src * _N + dst
    @compute_on("tpu_sparsecore")
    def _winner(kk, ee):
        return jnp.zeros((_N * _N,), jnp.int32).at[kk].max(ee)

    surv = _winner(key, e)[key] == e
    trash_off = (e & 8191) * 8
    shifted = jnp.where(key >= _HALF, key + _PAD, key)
    skey = jnp.where(surv, shifted, _TR0 + trash_off)

    S, O, Mmsg2 = _k1(inst_features, Ws, bs.reshape(1, -1), Wo,
                      bo.reshape(1, -1), Wmsg, bmsg.reshape(1, -1))
    Se, Oe = _sc_gather(S, O, src, dst, _E)
    gp1, m = _k2(union_features, Se, Oe,
                 surv.astype(jnp.float32).reshape(_E // 1024, 1, 1024),
                 Wu, bu.reshape(1, -1), Wg, bg.reshape(1, -1))
    A_flat = _sc_scatter(gp1.reshape(256, 128), skey.reshape(256, 128))
    A2d = A_flat.reshape(_FLATA // _N, _N)
    out1, out2, valid = _k4(A2d, Mmsg2, m)
    return _k5(out1, out2, valid, Wout1, bout1.reshape(1, -1),
               gamma.reshape(1, -1), beta.reshape(1, -1), Wout2,
               bout2.reshape(1, -1))
